# baseline (speedup 1.0000x reference)
import functools

import jax
import jax.numpy as jnp
from jax.experimental import pallas as pl

_BLOCK_ROWS = 1024


def _fused_mlp_body(x_ref, a0_ref, b0a_ref, b0_ref, b0b_ref, a1_ref,
                    b1pre_ref, p0_ref, p1_ref, b1post_ref, out_ref):
    x = x_ref[...]
    xa = x[:, 0:128]
    xb = x[:, 128:256]
    m0 = x[:, 256:257] != 0.0
    m1 = x[:, 257:258] != 0.0

    h = jnp.maximum(jnp.dot(xa, a0_ref[...]) + b0a_ref[...], 0.0)
    x0 = jnp.maximum(jnp.dot(h, b0_ref[...]) + b0b_ref[...], 0.0)
    x0 = jnp.where(m0, x0, 0.0)

    h1 = jnp.maximum(jnp.dot(xb, a1_ref[...]) + b1pre_ref[...], 0.0)

    out = jnp.dot(x0, p0_ref[...]) + jnp.dot(h1, p1_ref[...]) + b1post_ref[...]
    out_ref[...] = jnp.where(m1, out, 0.0)


@jax.jit
def kernel(input_val, W0a, b0a, W0b, b0b, W1pre, b1pre, W1post, b1post):
    n, d_in = input_val.shape
    block = min(_BLOCK_ROWS, n)
    grid = (n // block,)

    a0 = W0a.T
    b0 = W0b.T
    a1 = W1pre.T
    p0 = W1post[:, :128].T
    p1 = W1post[:, 128:].T

    row_spec = lambda w: pl.BlockSpec(w.shape, lambda i: (0, 0))
    out = pl.pallas_call(
        _fused_mlp_body,
        grid=grid,
        in_specs=[
            pl.BlockSpec((block, d_in), lambda i: (i, 0)),
            row_spec(a0),
            pl.BlockSpec((1, 256), lambda i: (0, 0)),
            row_spec(b0),
            pl.BlockSpec((1, 128), lambda i: (0, 0)),
            row_spec(a1),
            pl.BlockSpec((1, 256), lambda i: (0, 0)),
            row_spec(p0),
            row_spec(p1),
            pl.BlockSpec((1, 128), lambda i: (0, 0)),
        ],
        out_specs=pl.BlockSpec((block, 128), lambda i: (i, 0)),
        out_shape=jax.ShapeDtypeStruct((n, 128), input_val.dtype),
    )(input_val, a0, b0a.reshape(1, 256), b0, b0b.reshape(1, 128),
      a1, b1pre.reshape(1, 256), p0, p1, b1post.reshape(1, 128))
    return out


# speedup vs baseline: 1.1309x; 1.1309x over previous
import functools

import jax
import jax.numpy as jnp
from jax.experimental import pallas as pl

_BLOCK_ROWS = 4096


def _fused_mlp_body(x_ref, a0_ref, b0a_ref, b0_ref, b0b_ref, a1_ref,
                    b1pre_ref, p0_ref, p1_ref, b1post_ref, out_ref):
    x = x_ref[...]
    xa = x[:, 0:128]
    xb = x[:, 128:256]
    m0 = x[:, 256:257] != 0.0
    m1 = x[:, 257:258] != 0.0

    h = jnp.maximum(jnp.dot(xa, a0_ref[...]) + b0a_ref[...], 0.0)
    x0 = jnp.maximum(jnp.dot(h, b0_ref[...]) + b0b_ref[...], 0.0)
    x0 = jnp.where(m0, x0, 0.0)

    h1 = jnp.maximum(jnp.dot(xb, a1_ref[...]) + b1pre_ref[...], 0.0)

    out = jnp.dot(x0, p0_ref[...]) + jnp.dot(h1, p1_ref[...]) + b1post_ref[...]
    out_ref[...] = jnp.where(m1, out, 0.0)


@jax.jit
def kernel(input_val, W0a, b0a, W0b, b0b, W1pre, b1pre, W1post, b1post):
    n, d_in = input_val.shape
    block = min(_BLOCK_ROWS, n)
    grid = (n // block,)

    a0 = W0a.T
    b0 = W0b.T
    a1 = W1pre.T
    p0 = W1post[:, :128].T
    p1 = W1post[:, 128:].T

    row_spec = lambda w: pl.BlockSpec(w.shape, lambda i: (0, 0))
    out = pl.pallas_call(
        _fused_mlp_body,
        grid=grid,
        in_specs=[
            pl.BlockSpec((block, d_in), lambda i: (i, 0)),
            row_spec(a0),
            pl.BlockSpec((1, 256), lambda i: (0, 0)),
            row_spec(b0),
            pl.BlockSpec((1, 128), lambda i: (0, 0)),
            row_spec(a1),
            pl.BlockSpec((1, 256), lambda i: (0, 0)),
            row_spec(p0),
            row_spec(p1),
            pl.BlockSpec((1, 128), lambda i: (0, 0)),
        ],
        out_specs=pl.BlockSpec((block, 128), lambda i: (i, 0)),
        out_shape=jax.ShapeDtypeStruct((n, 128), input_val.dtype),
    )(input_val, a0, b0a.reshape(1, 256), b0, b0b.reshape(1, 128),
      a1, b1pre.reshape(1, 256), p0, p1, b1post.reshape(1, 128))
    return out
